# Initial kernel scaffold; baseline (speedup 1.0000x reference)
import jax
import jax.numpy as jnp
from jax.experimental import pallas as pl


def kernel(input_ids, input_ids_gram2, input_ids_gram3, input_mask, labels, emb_word, emb_g2, emb_g3, W1, b1, W2, b2):
    raise NotImplementedError("write your pallas kernel here")


# speedup vs baseline: 14.7632x; 14.7632x over previous
import functools

import jax
import jax.numpy as jnp
from jax import lax
from jax.experimental import pallas as pl
from jax.experimental.pallas import tpu as pltpu
from jax.experimental.pallas import tpu_sc as plsc


def _cdiv(a, b):
    return (a + b - 1) // b


def _rowsums_scaled(table, scale):
    n, d = table.shape
    blk = 2048

    def body(t_ref, o_ref):
        o_ref[...] = jnp.sum(t_ref[...], axis=1, keepdims=True) * scale

    return pl.pallas_call(
        body,
        grid=(_cdiv(n, blk),),
        in_specs=[pl.BlockSpec((blk, d), lambda i: (i, 0))],
        out_specs=pl.BlockSpec((blk, 1), lambda i: (i, 0)),
        out_shape=jax.ShapeDtypeStruct((n, 1), jnp.float32),
    )(table)


def _sc_gather_sum(rs_w, rs_2, rs_3, idx_w, idx_2, idx_3):
    info = plsc.get_sparse_core_info()
    nw = info.num_cores * info.num_subcores
    ntot = idx_w.shape[0]
    per_w = ntot // nw
    chunk = 16384
    nchunk = per_w // chunk
    assert ntot % nw == 0 and per_w % chunk == 0
    mesh = plsc.VectorSubcoreMesh(core_axis_name="c", subcore_axis_name="s")

    @functools.partial(
        pl.kernel,
        mesh=mesh,
        out_type=jax.ShapeDtypeStruct((ntot,), jnp.float32),
        scratch_types=[
            pltpu.VMEM((chunk,), jnp.int32),
            pltpu.VMEM((chunk,), jnp.float32),
            pltpu.VMEM((chunk,), jnp.float32),
            pltpu.VMEM((chunk,), jnp.float32),
            pltpu.SemaphoreType.DMA,
        ],
    )
    def k(rsw_h, rs2_h, rs3_h, iw_h, i2_h, i3_h, out_h,
          idx_v, vw_v, v2_v, v3_v, sem):
        wid = lax.axis_index("s") * info.num_cores + lax.axis_index("c")

        def per_chunk(c, _):
            base = wid * per_w + c * chunk
            pltpu.sync_copy(iw_h.at[pl.ds(base, chunk)], idx_v)
            pltpu.async_copy(rsw_h.at[idx_v], vw_v, sem).wait()
            pltpu.sync_copy(i2_h.at[pl.ds(base, chunk)], idx_v)
            pltpu.async_copy(rs2_h.at[idx_v], v2_v, sem).wait()
            pltpu.sync_copy(i3_h.at[pl.ds(base, chunk)], idx_v)
            pltpu.async_copy(rs3_h.at[idx_v], v3_v, sem).wait()

            def add_vec(i, _):
                s = pl.ds(i * 16, 16)
                vw_v[s] = vw_v[s] + v2_v[s] + v3_v[s]
                return 0

            lax.fori_loop(0, chunk // 16, add_vec, 0, unroll=8)
            pltpu.sync_copy(vw_v, out_h.at[pl.ds(base, chunk)])
            return 0

        lax.fori_loop(0, nchunk, per_chunk, 0)

    return k(rs_w, rs_2, rs_3, idx_w, idx_2, idx_3)


def _mlp_head(x, w1, b1, w2p, b2p):
    bsz, l = x.shape
    d = w1.shape[1]
    blk = 512

    def body(x_ref, w1_ref, b1_ref, w2_ref, b2_ref, o_ref):
        h = jnp.dot(x_ref[...], w1_ref[...], preferred_element_type=jnp.float32)
        h = jnp.maximum(h + b1_ref[...], 0.0)
        o_ref[...] = (
            jnp.dot(h, w2_ref[...], preferred_element_type=jnp.float32)
            + b2_ref[...]
        )

    return pl.pallas_call(
        body,
        grid=(bsz // blk,),
        in_specs=[
            pl.BlockSpec((blk, l), lambda i: (i, 0)),
            pl.BlockSpec((l, d), lambda i: (0, 0)),
            pl.BlockSpec((1, d), lambda i: (0, 0)),
            pl.BlockSpec((d, d), lambda i: (0, 0)),
            pl.BlockSpec((1, d), lambda i: (0, 0)),
        ],
        out_specs=pl.BlockSpec((blk, d), lambda i: (i, 0)),
        out_shape=jax.ShapeDtypeStruct((bsz, d), jnp.float32),
    )(x, w1, b1, w2p, b2p)


def kernel(input_ids, input_ids_gram2, input_ids_gram3, input_mask, labels,
           emb_word, emb_g2, emb_g3, W1, b1, W2, b2):
    bsz, l = input_ids.shape
    d = W1.shape[1]
    num_labels = W2.shape[1]
    scale = 1.0 / float(l)

    rs_w = _rowsums_scaled(emb_word, scale).reshape(-1)
    rs_2 = _rowsums_scaled(emb_g2, scale).reshape(-1)
    rs_3 = _rowsums_scaled(emb_g3, scale).reshape(-1)

    pooled = _sc_gather_sum(
        rs_w, rs_2, rs_3,
        input_ids.reshape(-1),
        input_ids_gram2.reshape(-1),
        input_ids_gram3.reshape(-1),
    ).reshape(bsz, l)

    w2p = jnp.zeros((d, d), jnp.float32).at[:, :num_labels].set(W2)
    b2p = jnp.zeros((1, d), jnp.float32).at[0, :num_labels].set(b2)
    out_full = _mlp_head(pooled, W1, b1.reshape(1, d), w2p, b2p)
    return out_full[:, :num_labels]


# speedup vs baseline: 17.1032x; 1.1585x over previous
import functools

import jax
import jax.numpy as jnp
from jax import lax
from jax.experimental import pallas as pl
from jax.experimental.pallas import tpu as pltpu
from jax.experimental.pallas import tpu_sc as plsc


def _cdiv(a, b):
    return (a + b - 1) // b


def _rowsums_scaled(table, scale):
    n, d = table.shape
    blk = 2048

    def body(t_ref, o_ref):
        o_ref[...] = jnp.sum(t_ref[...], axis=1, keepdims=True) * scale

    return pl.pallas_call(
        body,
        grid=(_cdiv(n, blk),),
        in_specs=[pl.BlockSpec((blk, d), lambda i: (i, 0))],
        out_specs=pl.BlockSpec((blk, 1), lambda i: (i, 0)),
        out_shape=jax.ShapeDtypeStruct((n, 1), jnp.float32),
    )(table)


def _sc_gather(rs, idx):
    info = plsc.get_sparse_core_info()
    nw = info.num_cores * info.num_subcores
    ntot = idx.shape[0]
    per_w = ntot // nw
    chunk = 16384
    nchunk = per_w // chunk
    assert ntot % nw == 0 and per_w % chunk == 0
    mesh = plsc.VectorSubcoreMesh(core_axis_name="c", subcore_axis_name="s")

    @functools.partial(
        pl.kernel,
        mesh=mesh,
        out_type=jax.ShapeDtypeStruct((ntot,), jnp.float32),
        scratch_types=[
            [pltpu.VMEM((chunk,), jnp.int32)] * 2,
            [pltpu.VMEM((chunk,), jnp.float32)] * 2,
            [pltpu.SemaphoreType.DMA] * 2,
            [pltpu.SemaphoreType.DMA] * 2,
        ],
    )
    def k(rs_h, idx_h, out_h, idx_v, val_v, gsem, osem):
        wid = lax.axis_index("s") * info.num_cores + lax.axis_index("c")
        gathers = [None] * nchunk
        stores = [None] * nchunk
        for c in range(nchunk + 1):
            if c < nchunk:
                b = c % 2
                base = wid * per_w + c * chunk
                if c >= 2:
                    stores[c - 2].wait()
                pltpu.sync_copy(idx_h.at[pl.ds(base, chunk)], idx_v[b])
                gathers[c] = pltpu.async_copy(rs_h.at[idx_v[b]], val_v[b],
                                              gsem[b])
            if c >= 1:
                d = c - 1
                pb = d % 2
                gathers[d].wait()
                dbase = wid * per_w + d * chunk
                stores[d] = pltpu.async_copy(
                    val_v[pb], out_h.at[pl.ds(dbase, chunk)], osem[pb])
        stores[nchunk - 2].wait()
        stores[nchunk - 1].wait()

    return k(rs, idx)


def _mlp_head(g1, g2, g3, w1, b1, w2p, b2p):
    bsz, l = g1.shape
    d = w1.shape[1]
    blk = 512

    def body(g1_ref, g2_ref, g3_ref, w1_ref, b1_ref, w2_ref, b2_ref, o_ref):
        x = g1_ref[...] + g2_ref[...] + g3_ref[...]
        h = jnp.dot(x, w1_ref[...], preferred_element_type=jnp.float32)
        h = jnp.maximum(h + b1_ref[...], 0.0)
        o_ref[...] = (
            jnp.dot(h, w2_ref[...], preferred_element_type=jnp.float32)
            + b2_ref[...]
        )

    xspec = pl.BlockSpec((blk, l), lambda i: (i, 0))
    return pl.pallas_call(
        body,
        grid=(bsz // blk,),
        in_specs=[
            xspec,
            xspec,
            xspec,
            pl.BlockSpec((l, d), lambda i: (0, 0)),
            pl.BlockSpec((1, d), lambda i: (0, 0)),
            pl.BlockSpec((d, d), lambda i: (0, 0)),
            pl.BlockSpec((1, d), lambda i: (0, 0)),
        ],
        out_specs=pl.BlockSpec((blk, d), lambda i: (i, 0)),
        out_shape=jax.ShapeDtypeStruct((bsz, d), jnp.float32),
    )(g1, g2, g3, w1, b1, w2p, b2p)


def kernel(input_ids, input_ids_gram2, input_ids_gram3, input_mask, labels,
           emb_word, emb_g2, emb_g3, W1, b1, W2, b2):
    bsz, l = input_ids.shape
    d = W1.shape[1]
    num_labels = W2.shape[1]
    scale = 1.0 / float(l)

    rs_w = _rowsums_scaled(emb_word, scale).reshape(-1)
    g_w = _sc_gather(rs_w, input_ids.reshape(-1)).reshape(bsz, l)
    rs_2 = _rowsums_scaled(emb_g2, scale).reshape(-1)
    g_2 = _sc_gather(rs_2, input_ids_gram2.reshape(-1)).reshape(bsz, l)
    rs_3 = _rowsums_scaled(emb_g3, scale).reshape(-1)
    g_3 = _sc_gather(rs_3, input_ids_gram3.reshape(-1)).reshape(bsz, l)

    w2p = jnp.zeros((d, d), jnp.float32).at[:, :num_labels].set(W2)
    b2p = jnp.zeros((1, d), jnp.float32).at[0, :num_labels].set(b2)
    out_full = _mlp_head(g_w, g_2, g_3, W1, b1.reshape(1, d), w2p, b2p)
    return out_full[:, :num_labels]
